# baseline (speedup 1.0000x reference)
import jax
import jax.numpy as jnp
from jax.experimental import pallas as pl
from jax.experimental.pallas import tpu as pltpu

B, S, D, H = 2, 2048, 2048, 16
DH = D // H
HALF = DH // 2
BS = B * S
ROPE_THETA = 10000.0
LN_EPS = 1e-5


BM1 = 1024
BN1 = 1024
NSEC = D // BN1


def _ln_qkv_rope_kernel(x_ref, w_ref, cs_ref, q_ref, k_ref, v_ref, xn_ref):
    j = pl.program_id(1)

    @pl.when(j == 0)
    def _():
        xb = x_ref[...]
        mu = jnp.mean(xb, axis=-1, keepdims=True)
        xc = xb - mu
        var = jnp.mean(xc * xc, axis=-1, keepdims=True)
        xn_ref[...] = (xc * jax.lax.rsqrt(var + LN_EPS)).astype(jnp.bfloat16)

    y = jnp.dot(xn_ref[...], w_ref[...], preferred_element_type=jnp.float32)

    def rope(y):
        cos = cs_ref[:, :HALF]
        sin = cs_ref[:, HALF:]
        cosf = jnp.concatenate([cos, cos], axis=-1)
        sinf = jnp.concatenate([-sin, sin], axis=-1)
        parts = []
        for h in range(BN1 // DH):
            yh = y[:, h * DH:(h + 1) * DH]
            rot = pltpu.roll(yh, HALF, axis=1)
            parts.append(yh * cosf + rot * sinf)
        return jnp.concatenate(parts, axis=-1).astype(jnp.bfloat16)

    @pl.when(j < NSEC)
    def _():
        q_ref[0] = rope(y)

    @pl.when(jnp.logical_and(j >= NSEC, j < 2 * NSEC))
    def _():
        k_ref[0] = rope(y)

    @pl.when(j >= 2 * NSEC)
    def _():
        v_ref[0] = y.astype(jnp.bfloat16)


def _ln_qkv_rope(x2, w_qkv, cs):
    nsb = S // BM1
    sds = jax.ShapeDtypeStruct((B, S, D), jnp.bfloat16)

    def out_idx(lo):
        def f(i, j):
            return (i // nsb, i % nsb, jnp.clip(j - lo, 0, NSEC - 1))
        return f

    return pl.pallas_call(
        _ln_qkv_rope_kernel,
        out_shape=(sds, sds, sds),
        grid=(BS // BM1, (3 * D) // BN1),
        in_specs=[
            pl.BlockSpec((BM1, D), lambda i, j: (i, 0)),
            pl.BlockSpec((D, BN1), lambda i, j: (0, j)),
            pl.BlockSpec((BM1, DH), lambda i, j: (i, 0)),
        ],
        out_specs=(
            pl.BlockSpec((1, BM1, BN1), out_idx(0)),
            pl.BlockSpec((1, BM1, BN1), out_idx(NSEC)),
            pl.BlockSpec((1, BM1, BN1), out_idx(2 * NSEC)),
        ),
        scratch_shapes=[pltpu.VMEM((BM1, D), jnp.bfloat16)],
        compiler_params=pltpu.CompilerParams(
            dimension_semantics=("parallel", "arbitrary"),
            allow_input_fusion=(False, True, False),
            vmem_limit_bytes=56 * 1024 * 1024,
            skip_device_barrier=True,
        ),
        name="ln_qkv_rope",
    )(x2, w_qkv, cs)


BQ = 512
BK = 512
NSPLIT = 2
BKV = BK * NSPLIT
NKB = S // BKV
LOG2E = 1.4426950408889634
NEG = -1e30


NCH = S // BK


def _attn_kernel(q_ref, k_ref, v_ref, o_ref, pv_s, m_s, l_s):
    qi = pl.program_id(1)
    kb = pl.program_id(2)

    @pl.when(jnp.logical_and(pl.program_id(0) == 0,
                             jnp.logical_and(qi == 0, kb == 0)))
    def _():
        pv_s[...] = jnp.zeros_like(pv_s)
        m_s[...] = jnp.full_like(m_s, NEG)
        l_s[...] = jnp.zeros_like(l_s)

    def block(masked):
        qs = q_ref[0] * jnp.bfloat16(DH ** -0.5 * LOG2E)
        for c in range(NSPLIT):
            slot = kb * NSPLIT + c
            k = k_ref[0, c * BK:(c + 1) * BK, :]
            v = v_ref[0, c * BK:(c + 1) * BK, :]
            s = jax.lax.dot_general(
                qs, k, (((1,), (1,)), ((), ())),
                preferred_element_type=jnp.float32)
            if masked:
                rows = qi * BQ + jax.lax.broadcasted_iota(
                    jnp.int32, (BQ, 1), 0)
                cols = (kb * BKV + c * BK
                        + jax.lax.broadcasted_iota(jnp.int32, (1, BK), 1))
                s = jnp.where(rows >= cols, s, NEG)
            mc = jnp.max(s, axis=-1, keepdims=True)
            p = jnp.exp2(s - mc)
            lc = jnp.sum(p, axis=-1, keepdims=True)
            pv = jnp.dot(p.astype(jnp.bfloat16), v,
                         preferred_element_type=jnp.float32)
            pv_s[slot] = pv
            m_s[slot] = jnp.broadcast_to(mc, (BQ, 128))
            l_s[slot] = jnp.broadcast_to(lc, (BQ, 128))

    dkb = qi // (BKV // BQ)

    @pl.when(kb < dkb)
    def _():
        block(masked=False)

    @pl.when(kb == dkb)
    def _():
        block(masked=True)
        mts = [jnp.where(t // NSPLIT <= dkb, m_s[t, :, 0:1], NEG)
               for t in range(NCH)]
        m_star = mts[0]
        for t in range(1, NCH):
            m_star = jnp.maximum(m_star, mts[t])
        acc = pv_s[0] * jnp.exp2(mts[0] - m_star)
        l = l_s[0, :, 0:1] * jnp.exp2(mts[0] - m_star)
        for t in range(1, NCH):
            a = jnp.exp2(mts[t] - m_star)
            acc = acc + pv_s[t] * a
            l = l + l_s[t, :, 0:1] * a
        o_ref[0] = (acc / l).astype(jnp.bfloat16)


def _flash_attn(q3, k3, v3):
    qb = BKV // BQ

    def kv_idx(bh, qi, kb):
        return (bh // H, jnp.minimum(kb, qi // qb), bh % H)

    return pl.pallas_call(
        _attn_kernel,
        out_shape=jax.ShapeDtypeStruct((B, S, D), jnp.bfloat16),
        grid=(B * H, S // BQ, NKB),
        in_specs=[
            pl.BlockSpec((1, BQ, DH),
                         lambda bh, qi, kb: (bh // H, qi, bh % H)),
            pl.BlockSpec((1, BKV, DH), kv_idx),
            pl.BlockSpec((1, BKV, DH), kv_idx),
        ],
        out_specs=pl.BlockSpec((1, BQ, DH),
                               lambda bh, qi, kb: (bh // H, qi, bh % H)),
        scratch_shapes=[
            pltpu.VMEM((NCH, BQ, DH), jnp.float32),
            pltpu.VMEM((NCH, BQ, 128), jnp.float32),
            pltpu.VMEM((NCH, BQ, 128), jnp.float32),
        ],
        compiler_params=pltpu.CompilerParams(
            dimension_semantics=("parallel", "parallel", "arbitrary"),
            skip_device_barrier=True,
        ),
        name="flash_attn",
    )(q3, k3, v3)


BM3 = 512


def _proj_kernel(x_ref, w_ref, o_ref):
    o_ref[...] = jnp.dot(x_ref[...], w_ref[...],
                         preferred_element_type=jnp.float32)


def _out_proj(x2, w_out):
    return pl.pallas_call(
        _proj_kernel,
        out_shape=jax.ShapeDtypeStruct((BS, D), jnp.float32),
        grid=(BS // BM3,),
        in_specs=[
            pl.BlockSpec((BM3, D), lambda i: (i, 0)),
            pl.BlockSpec((D, D), lambda i: (0, 0)),
        ],
        out_specs=pl.BlockSpec((BM3, D), lambda i: (i, 0)),
        compiler_params=pltpu.CompilerParams(
            dimension_semantics=("parallel",),
            allow_input_fusion=(False, True),
            vmem_limit_bytes=50 * 1024 * 1024,
            skip_device_barrier=True,
        ),
        name="out_proj",
    )(x2, w_out)


def kernel(positions, hidden_states, w_qkv, w_out):
    x2 = hidden_states.reshape(BS, D)
    pos_f = positions.reshape(BS).astype(jnp.float32)
    inv_freq = 1.0 / (ROPE_THETA ** (
        jnp.arange(HALF, dtype=jnp.float32) / HALF))
    ang = pos_f[:, None] * inv_freq[None, :]
    cs = jnp.concatenate([jnp.cos(ang), jnp.sin(ang)], axis=-1)

    q3, k3, v3 = _ln_qkv_rope(x2, w_qkv.astype(jnp.bfloat16), cs)

    attn = _flash_attn(q3, k3, v3)
    out = _out_proj(attn.reshape(BS, D), w_out.astype(jnp.bfloat16))
    return out.reshape(B, S, D)


# speedup vs baseline: 1.0082x; 1.0082x over previous
import jax
import jax.numpy as jnp
from jax.experimental import pallas as pl
from jax.experimental.pallas import tpu as pltpu

B, S, D, H = 2, 2048, 2048, 16
DH = D // H
HALF = DH // 2
BS = B * S
ROPE_THETA = 10000.0
LN_EPS = 1e-5


BM1 = 1024
BN1 = 1024
NSEC = D // BN1


def _ln_qkv_rope_kernel(x_ref, w_ref, cs_ref, o_ref, xn_ref):
    j = pl.program_id(1)

    @pl.when(j == 0)
    def _():
        xb = x_ref[...]
        mu = jnp.mean(xb, axis=-1, keepdims=True)
        xc = xb - mu
        var = jnp.mean(xc * xc, axis=-1, keepdims=True)
        xn_ref[...] = (xc * jax.lax.rsqrt(var + LN_EPS)).astype(jnp.bfloat16)

    def rope(y):
        cos = cs_ref[:, :HALF]
        sin = cs_ref[:, HALF:]
        cosf = jnp.concatenate([cos, cos], axis=-1)
        sinf = jnp.concatenate([-sin, sin], axis=-1)
        parts = []
        for h in range(BN1 // DH):
            yh = y[:, h * DH:(h + 1) * DH]
            rot = pltpu.roll(yh, HALF, axis=1)
            parts.append(yh * cosf + rot * sinf)
        return jnp.concatenate(parts, axis=-1).astype(jnp.bfloat16)

    for half in range(D // BN1):
        sl = slice(half * BN1, (half + 1) * BN1)
        y = jnp.dot(xn_ref[...], w_ref[:, sl],
                    preferred_element_type=jnp.float32)

        @pl.when(j < 2)
        def _(y=y, sl=sl):
            o_ref[0, 0, :, sl] = rope(y)

        @pl.when(j >= 2)
        def _(y=y, sl=sl):
            o_ref[0, 0, :, sl] = y.astype(jnp.bfloat16)


def _ln_qkv_rope(x2, w_qkv, cs):
    nsb = S // BM1

    return pl.pallas_call(
        _ln_qkv_rope_kernel,
        out_shape=jax.ShapeDtypeStruct((3, B, S, D), jnp.bfloat16),
        grid=(BS // BM1, 3),
        in_specs=[
            pl.BlockSpec((BM1, D), lambda i, j: (i, 0)),
            pl.BlockSpec((D, D), lambda i, j: (0, j)),
            pl.BlockSpec((BM1, DH), lambda i, j: (i, 0)),
        ],
        out_specs=pl.BlockSpec(
            (1, 1, BM1, D), lambda i, j: (j, i // nsb, i % nsb, 0)),
        scratch_shapes=[pltpu.VMEM((BM1, D), jnp.bfloat16)],
        compiler_params=pltpu.CompilerParams(
            dimension_semantics=("parallel", "arbitrary"),
            allow_input_fusion=(False, True, False),
            vmem_limit_bytes=56 * 1024 * 1024,
            skip_device_barrier=True,
        ),
        name="ln_qkv_rope",
    )(x2, w_qkv, cs)


BQ = 512
BK = 512
NSPLIT = 2
BKV = BK * NSPLIT
NKB = S // BKV
LOG2E = 1.4426950408889634
NEG = -1e30


NCH = S // BK


def _attn_kernel(q_ref, k_ref, v_ref, o_ref, pv_s, m_s, l_s):
    qi = pl.program_id(1)
    kb = pl.program_id(2)

    @pl.when(jnp.logical_and(pl.program_id(0) == 0,
                             jnp.logical_and(qi == 0, kb == 0)))
    def _():
        pv_s[...] = jnp.zeros_like(pv_s)
        m_s[...] = jnp.full_like(m_s, NEG)
        l_s[...] = jnp.zeros_like(l_s)

    def block(masked):
        qs = q_ref[0, 0] * jnp.bfloat16(DH ** -0.5 * LOG2E)
        for c in range(NSPLIT):
            slot = kb * NSPLIT + c
            k = k_ref[0, 0, c * BK:(c + 1) * BK, :]
            v = v_ref[0, 0, c * BK:(c + 1) * BK, :]
            s = jax.lax.dot_general(
                qs, k, (((1,), (1,)), ((), ())),
                preferred_element_type=jnp.float32)
            if masked:
                rows = qi * BQ + jax.lax.broadcasted_iota(
                    jnp.int32, (BQ, 1), 0)
                cols = (kb * BKV + c * BK
                        + jax.lax.broadcasted_iota(jnp.int32, (1, BK), 1))
                s = jnp.where(rows >= cols, s, NEG)
            mc = jnp.max(s, axis=-1, keepdims=True)
            p = jnp.exp2(s - mc)
            lc = jnp.sum(p, axis=-1, keepdims=True)
            pv = jnp.dot(p.astype(jnp.bfloat16), v,
                         preferred_element_type=jnp.float32)
            pv_s[slot] = pv
            m_s[slot] = jnp.broadcast_to(mc, (BQ, 128))
            l_s[slot] = jnp.broadcast_to(lc, (BQ, 128))

    dkb = qi // (BKV // BQ)

    @pl.when(kb < dkb)
    def _():
        block(masked=False)

    @pl.when(kb == dkb)
    def _():
        block(masked=True)
        mts = [jnp.where(t // NSPLIT <= dkb, m_s[t, :, 0:1], NEG)
               for t in range(NCH)]
        m_star = mts[0]
        for t in range(1, NCH):
            m_star = jnp.maximum(m_star, mts[t])
        acc = pv_s[0] * jnp.exp2(mts[0] - m_star)
        l = l_s[0, :, 0:1] * jnp.exp2(mts[0] - m_star)
        for t in range(1, NCH):
            a = jnp.exp2(mts[t] - m_star)
            acc = acc + pv_s[t] * a
            l = l + l_s[t, :, 0:1] * a
        o_ref[0] = (acc / l).astype(jnp.bfloat16)


def _flash_attn(qkv3):
    qb = BKV // BQ

    def kv_idx(which):
        def f(bh, qi, kb):
            return (which, bh // H, jnp.minimum(kb, qi // qb), bh % H)
        return f

    return pl.pallas_call(
        _attn_kernel,
        out_shape=jax.ShapeDtypeStruct((B, S, D), jnp.bfloat16),
        grid=(B * H, S // BQ, NKB),
        in_specs=[
            pl.BlockSpec((1, 1, BQ, DH),
                         lambda bh, qi, kb: (0, bh // H, qi, bh % H)),
            pl.BlockSpec((1, 1, BKV, DH), kv_idx(1)),
            pl.BlockSpec((1, 1, BKV, DH), kv_idx(2)),
        ],
        out_specs=pl.BlockSpec((1, BQ, DH),
                               lambda bh, qi, kb: (bh // H, qi, bh % H)),
        scratch_shapes=[
            pltpu.VMEM((NCH, BQ, DH), jnp.float32),
            pltpu.VMEM((NCH, BQ, 128), jnp.float32),
            pltpu.VMEM((NCH, BQ, 128), jnp.float32),
        ],
        compiler_params=pltpu.CompilerParams(
            dimension_semantics=("parallel", "parallel", "arbitrary"),
            skip_device_barrier=True,
        ),
        name="flash_attn",
    )(qkv3, qkv3, qkv3)


BM3 = 512


def _proj_kernel(x_ref, w_ref, o_ref):
    o_ref[...] = jnp.dot(x_ref[...], w_ref[...],
                         preferred_element_type=jnp.float32)


def _out_proj(x2, w_out):
    return pl.pallas_call(
        _proj_kernel,
        out_shape=jax.ShapeDtypeStruct((BS, D), jnp.float32),
        grid=(BS // BM3,),
        in_specs=[
            pl.BlockSpec((BM3, D), lambda i: (i, 0)),
            pl.BlockSpec((D, D), lambda i: (0, 0)),
        ],
        out_specs=pl.BlockSpec((BM3, D), lambda i: (i, 0)),
        compiler_params=pltpu.CompilerParams(
            dimension_semantics=("parallel",),
            allow_input_fusion=(False, True),
            vmem_limit_bytes=50 * 1024 * 1024,
            skip_device_barrier=True,
        ),
        name="out_proj",
    )(x2, w_out)


def kernel(positions, hidden_states, w_qkv, w_out):
    x2 = hidden_states.reshape(BS, D)
    pos_f = positions.reshape(BS).astype(jnp.float32)
    inv_freq = 1.0 / (ROPE_THETA ** (
        jnp.arange(HALF, dtype=jnp.float32) / HALF))
    ang = pos_f[:, None] * inv_freq[None, :]
    cs = jnp.concatenate([jnp.cos(ang), jnp.sin(ang)], axis=-1)

    qkv3 = _ln_qkv_rope(x2, w_qkv.astype(jnp.bfloat16), cs)

    attn = _flash_attn(qkv3)
    out = _out_proj(attn.reshape(BS, D), w_out.astype(jnp.bfloat16))
    return out.reshape(B, S, D)
